# Initial kernel scaffold; baseline (speedup 1.0000x reference)
import jax
import jax.numpy as jnp
from jax.experimental import pallas as pl


def kernel(feature_value, tables):
    raise NotImplementedError("write your pallas kernel here")


# speedup vs baseline: 1.1188x; 1.1188x over previous
import functools

import jax
import jax.numpy as jnp
from jax import lax
from jax.experimental import pallas as pl
from jax.experimental.pallas import tpu as pltpu
from jax.experimental.pallas import tpu_sc as plsc

_NUM_FIELDS = 26
_VOCAB = 100000
_DIM = 32
_BATCH = 16384

_NC = 2
_NS = 16
_NW = _NC * _NS
_BPW = _BATCH // _NW
_CHUNK = 128
_NCHUNK = _BPW // _CHUNK


def _gather_body(fv_hbm, tbl_hbm, out_hbm, idx_v, rows_v, sem):
    wid = lax.axis_index("s") * _NC + lax.axis_index("c")
    base = wid * _BPW

    def field_body(f, carry):
        pltpu.sync_copy(fv_hbm.at[f, pl.ds(base, _BPW)], idx_v)
        off = jnp.full((16,), f * _VOCAB, dtype=jnp.int32)

        def add_body(i, c):
            sl = pl.ds(i * 16, 16)
            idx_v[sl] = idx_v[sl] + off
            return c

        lax.fori_loop(0, _BPW // 16, add_body, 0)

        out_base = f * _BATCH + base

        def chunk_body(j, c):
            idx_slice = idx_v.at[pl.ds(j * _CHUNK, _CHUNK)]
            pltpu.async_copy(tbl_hbm.at[idx_slice], rows_v, sem).wait()
            pltpu.sync_copy(rows_v, out_hbm.at[pl.ds(out_base + j * _CHUNK, _CHUNK)])
            return c

        lax.fori_loop(0, _NCHUNK, chunk_body, 0)
        return carry

    lax.fori_loop(0, _NUM_FIELDS, field_body, 0)


@jax.jit
def _gather(fv_t, tbl_flat):
    mesh = plsc.VectorSubcoreMesh(core_axis_name="c", subcore_axis_name="s")
    return pl.kernel(
        _gather_body,
        mesh=mesh,
        out_type=jax.ShapeDtypeStruct((_NUM_FIELDS * _BATCH, _DIM), jnp.float32),
        scratch_types=[
            pltpu.VMEM((_BPW,), jnp.int32),
            pltpu.VMEM((_CHUNK, _DIM), jnp.float32),
            pltpu.SemaphoreType.DMA,
        ],
        compiler_params=pltpu.CompilerParams(use_tc_tiling_on_sc=False),
    )(fv_t, tbl_flat)


def kernel(feature_value, tables):
    fv_t = feature_value.T
    tbl_flat = tables.reshape(_NUM_FIELDS * _VOCAB, _DIM)
    out = _gather(fv_t, tbl_flat)
    return out.reshape(_NUM_FIELDS, _BATCH, _DIM)


# speedup vs baseline: 1.1859x; 1.0599x over previous
import functools

import jax
import jax.numpy as jnp
from jax import lax
from jax.experimental import pallas as pl
from jax.experimental.pallas import tpu as pltpu
from jax.experimental.pallas import tpu_sc as plsc

_NUM_FIELDS = 26
_VOCAB = 100000
_DIM = 32
_BATCH = 16384

_NC = 2
_NS = 16
_NW = _NC * _NS
_BPW = _BATCH // _NW
_CHUNK = 128
_NCHUNK = _BPW // _CHUNK


def _gather_body(fv_hbm, tbl_hbm, out_hbm,
                 idx0, idx1, rows0, rows1, gs0, gs1, os0, os1):
    wid = lax.axis_index("s") * _NC + lax.axis_index("c")
    base = wid * _BPW

    def load_fire(f, idx_v, rows_v, gsem):
        pltpu.sync_copy(fv_hbm.at[f, pl.ds(base, _BPW)], idx_v)
        off = jnp.full((16,), f * _VOCAB, dtype=jnp.int32)

        def add_body(i, c):
            sl = pl.ds(i * 16, 16)
            idx_v[sl] = idx_v[sl] + off
            return c

        lax.fori_loop(0, _BPW // 16, add_body, 0)
        for j in range(_NCHUNK):
            sl = pl.ds(j * _CHUNK, _CHUNK)
            pltpu.async_copy(tbl_hbm.at[idx_v.at[sl]], rows_v.at[sl], gsem)

    def drain_fire_out(f, idx_v, rows_v, gsem, osem):
        for j in range(_NCHUNK):
            sl = pl.ds(j * _CHUNK, _CHUNK)
            pltpu.make_async_copy(tbl_hbm.at[idx_v.at[sl]], rows_v.at[sl], gsem).wait()
        pltpu.async_copy(rows_v, out_hbm.at[pl.ds(f * _BATCH + base, _BPW)], osem)

    def wait_out(f, rows_v, osem):
        pltpu.make_async_copy(rows_v, out_hbm.at[pl.ds(f * _BATCH + base, _BPW)], osem).wait()

    load_fire(0, idx0, rows0, gs0)
    load_fire(1, idx1, rows1, gs1)

    def loop_body(i, c):
        f0 = 2 * i
        f1 = f0 + 1
        drain_fire_out(f0, idx0, rows0, gs0, os0)
        drain_fire_out(f1, idx1, rows1, gs1, os1)

        @pl.when(i < _NUM_FIELDS // 2 - 1)
        def _():
            wait_out(f0, rows0, os0)
            load_fire(f0 + 2, idx0, rows0, gs0)
            wait_out(f1, rows1, os1)
            load_fire(f1 + 2, idx1, rows1, gs1)

        return c

    lax.fori_loop(0, _NUM_FIELDS // 2, loop_body, 0)
    wait_out(_NUM_FIELDS - 2, rows0, os0)
    wait_out(_NUM_FIELDS - 1, rows1, os1)


@jax.jit
def _gather(fv_t, tbl_flat):
    mesh = plsc.VectorSubcoreMesh(core_axis_name="c", subcore_axis_name="s")
    return pl.kernel(
        _gather_body,
        mesh=mesh,
        out_type=jax.ShapeDtypeStruct((_NUM_FIELDS * _BATCH, _DIM), jnp.float32),
        scratch_types=[
            pltpu.VMEM((_BPW,), jnp.int32),
            pltpu.VMEM((_BPW,), jnp.int32),
            pltpu.VMEM((_BPW, _DIM), jnp.float32),
            pltpu.VMEM((_BPW, _DIM), jnp.float32),
            pltpu.SemaphoreType.DMA,
            pltpu.SemaphoreType.DMA,
            pltpu.SemaphoreType.DMA,
            pltpu.SemaphoreType.DMA,
        ],
        compiler_params=pltpu.CompilerParams(use_tc_tiling_on_sc=False),
    )(fv_t, tbl_flat)


def kernel(feature_value, tables):
    fv_t = feature_value.T
    tbl_flat = tables.reshape(_NUM_FIELDS * _VOCAB, _DIM)
    out = _gather(fv_t, tbl_flat)
    return out.reshape(_NUM_FIELDS, _BATCH, _DIM)
